# Initial kernel scaffold; baseline (speedup 1.0000x reference)
import jax
import jax.numpy as jnp
from jax.experimental import pallas as pl


def kernel(x, edge_index, edge_attr, W1, b1, W2, b2):
    raise NotImplementedError("write your pallas kernel here")


# speedup vs baseline: 2.0142x; 2.0142x over previous
import functools

import jax
import jax.numpy as jnp
from jax import lax
from jax.experimental import pallas as pl
from jax.experimental.pallas import tpu as pltpu
from jax.experimental.pallas import tpu_sc as plsc

N_NODES = 10000
N_EDGES = 320000
D = 128
D_EDGE = 16

NC = 2
NS = 16
CHUNK = 80
N_CHUNKS = N_EDGES // CHUNK
CH_PER_TILE = N_CHUNKS // NS
HALF = N_NODES // NC
DEG_ROWS = HALF // D + 1
ACC_ROWS = HALF + DEG_ROWS
NL = 16


def _node_matmul(x, W1a, W1b, b1):
    def body(x_ref, wa_ref, wb_ref, b1_ref, a_ref, bb_ref):
        xv = x_ref[...]
        a_ref[...] = jnp.dot(xv, wa_ref[...], preferred_element_type=jnp.float32)
        bb_ref[...] = (jnp.dot(xv, wb_ref[...], preferred_element_type=jnp.float32)
                       + b1_ref[...])
    return pl.pallas_call(
        body,
        out_shape=(jax.ShapeDtypeStruct((N_NODES, D), jnp.float32),
                   jax.ShapeDtypeStruct((N_NODES, D), jnp.float32)),
    )(x, W1a, W1b, b1.reshape(1, D))


_EBLK = 8000


def _edge_matmul(ea, W1c):
    def body(ea_ref, w_ref, c_ref):
        c_ref[...] = jnp.dot(ea_ref[...], w_ref[...],
                             preferred_element_type=jnp.float32)
    return pl.pallas_call(
        body,
        grid=(N_EDGES // _EBLK,),
        in_specs=[pl.BlockSpec((_EBLK, D_EDGE), lambda i: (i, 0)),
                  pl.BlockSpec((D_EDGE, D), lambda i: (0, 0))],
        out_specs=pl.BlockSpec((_EBLK, D), lambda i: (i, 0)),
        out_shape=jax.ShapeDtypeStruct((N_EDGES, D), jnp.float32),
    )(ea, W1c)


_NBLK = 2000


def _final_matmul(S, deg, W2, b2):
    def body(s_ref, d_ref, w_ref, b2_ref, o_ref):
        o_ref[...] = (jnp.dot(s_ref[...], w_ref[...],
                              preferred_element_type=jnp.float32)
                      + d_ref[...] * b2_ref[...])
    return pl.pallas_call(
        body,
        grid=(N_NODES // _NBLK,),
        in_specs=[pl.BlockSpec((_NBLK, D), lambda i: (i, 0)),
                  pl.BlockSpec((_NBLK, 1), lambda i: (i, 0)),
                  pl.BlockSpec((D, D), lambda i: (0, 0)),
                  pl.BlockSpec((1, D), lambda i: (0, 0))],
        out_specs=pl.BlockSpec((_NBLK, D), lambda i: (i, 0)),
        out_shape=jax.ShapeDtypeStruct((N_NODES, D), jnp.float32),
    )(S, deg, W2, b2.reshape(1, D))


def _sc_gather_relu_scatter(A, Bb, C, src, dst):
    mesh = plsc.VectorSubcoreMesh(core_axis_name="c", subcore_axis_name="s")

    @functools.partial(
        pl.kernel,
        mesh=mesh,
        out_type=(jax.ShapeDtypeStruct((N_NODES, D), jnp.float32),
                  jax.ShapeDtypeStruct((NC, DEG_ROWS, D), jnp.float32)),
        scratch_types=[
            pltpu.VMEM((CHUNK,), jnp.int32),
            pltpu.VMEM((CHUNK,), jnp.int32),
            pltpu.VMEM((CHUNK,), jnp.int32),
            pltpu.VMEM((CHUNK,), jnp.int32),
            pltpu.VMEM((CHUNK,), jnp.int32),
            pltpu.VMEM((CHUNK, D), jnp.float32),
            pltpu.VMEM((CHUNK, D), jnp.float32),
            pltpu.VMEM((CHUNK, D), jnp.float32),
            pltpu.VMEM((CHUNK, D), jnp.float32),
            pltpu.VMEM_SHARED((ACC_ROWS, D), jnp.float32),
            pltpu.SemaphoreType.DMA,
            pltpu.SemaphoreType.DMA,
            pltpu.SemaphoreType.DMA,
        ],
    )
    def k(a_hbm, b_hbm, c_hbm, src_hbm, dst_hbm, out_hbm, deg_hbm,
          src_v, dst_v, sidx_v, inb_v, pos_v, a_v, b_v, c_v, m_v, s_sh,
          sem_a, sem_b, sem_c):
        cid = lax.axis_index("c")
        sid = lax.axis_index("s")

        zero16 = jnp.zeros((NL,), jnp.float32)
        lane = lax.iota(jnp.int32, NL)
        lanes = [lane + NL * j for j in range(D // NL)]

        def zrow_body(e, carry):
            for j in range(D // NL):
                m_v[e, pl.ds(j * NL, NL)] = zero16
            return carry
        lax.fori_loop(0, CHUNK, zrow_body, 0)
        for off, rows in ((0, 80), (80, 80), (160, 80), (240, 75)):
            pltpu.sync_copy(m_v.at[pl.ds(0, rows)],
                            s_sh.at[pl.ds(sid * 315 + off, rows)])

        plsc.subcore_barrier()

        lo = cid * HALF
        lo2 = (1 - cid) * HALF

        def chunk_body(kk, carry):
            base = (sid + kk * NS) * CHUNK
            pltpu.sync_copy(src_hbm.at[pl.ds(base, CHUNK)], src_v)
            pltpu.sync_copy(dst_hbm.at[pl.ds(base, CHUNK)], dst_v)
            cp_a = pltpu.async_copy(a_hbm.at[src_v], a_v, sem_a)
            cp_b = pltpu.async_copy(b_hbm.at[dst_v], b_v, sem_b)
            cp_c = pltpu.async_copy(c_hbm.at[pl.ds(base, CHUNK)], c_v, sem_c)
            cp_a.wait()
            cp_b.wait()
            cp_c.wait()

            for g in range(CHUNK // NL):
                d16 = dst_v[pl.ds(g * NL, NL)]
                r16 = d16 - lo
                in_half = (r16 >= 0) & (r16 < HALF)
                r2 = d16 - lo2
                sidx_v[pl.ds(g * NL, NL)] = jnp.where(
                    in_half, r16, HALF + lax.shift_right_logical(r2, 7))
                inb_v[pl.ds(g * NL, NL)] = jnp.where(in_half, 1, 0)
                pos_v[pl.ds(g * NL, NL)] = lax.bitwise_and(r2, 127)

            def group_body(g, c2):
                f16 = inb_v[pl.ds(g * NL, NL)]
                p16 = pos_v[pl.ds(g * NL, NL)]
                for l in range(NL):
                    e = g * NL + l
                    flag = f16[l] > 0
                    pos = p16[l]
                    for j in range(D // NL):
                        sl = pl.ds(j * NL, NL)
                        msg = jnp.maximum(
                            a_v[e, sl] + b_v[e, sl] + c_v[e, sl], 0.0)
                        oh = jnp.where(lanes[j] == pos, 1.0, 0.0)
                        m_v[e, sl] = jnp.where(flag, msg, oh)
                return c2
            lax.fori_loop(0, CHUNK // NL, group_body, 0)

            pltpu.sync_copy(m_v, s_sh.at[sidx_v], add=True)
            return carry
        lax.fori_loop(0, CH_PER_TILE, chunk_body, 0)

        plsc.subcore_barrier()

        @pl.when(sid == 0)
        def _():
            pltpu.sync_copy(s_sh.at[pl.ds(0, HALF)],
                            out_hbm.at[pl.ds(cid * HALF, HALF)])
            pltpu.sync_copy(s_sh.at[pl.ds(HALF, DEG_ROWS)], deg_hbm.at[cid])

    return k(A, Bb, C, src, dst)


def kernel(x, edge_index, edge_attr, W1, b1, W2, b2):
    src = edge_index[0].astype(jnp.int32)
    dst = edge_index[1].astype(jnp.int32)
    W1a = W1[:D]
    W1b = W1[D:2 * D]
    W1c = W1[2 * D:]

    A, Bb = _node_matmul(x, W1a, W1b, b1)
    C = _edge_matmul(edge_attr, W1c)
    S, degr = _sc_gather_relu_scatter(A, Bb, C, src, dst)

    deg = jnp.concatenate([degr[1].reshape(-1)[:HALF],
                           degr[0].reshape(-1)[:HALF]])
    return _final_matmul(S, deg.reshape(N_NODES, 1), W2, b2)


# speedup vs baseline: 2.9055x; 1.4425x over previous
import functools

import jax
import jax.numpy as jnp
from jax import lax
from jax.experimental import pallas as pl
from jax.experimental.pallas import tpu as pltpu
from jax.experimental.pallas import tpu_sc as plsc

N_NODES = 10000
N_EDGES = 320000
D = 128
D_EDGE = 16

NC = 2
NS = 16
CHUNK = 80
N_CHUNKS = N_EDGES // CHUNK
CH_PER_TILE = N_CHUNKS // NS
HALF = N_NODES // NC
DEG_ROWS = HALF // D + 1
ACC_ROWS = HALF + DEG_ROWS
NL = 16


def _node_matmul(x, W1a, W1b, b1):
    def body(x_ref, wa_ref, wb_ref, b1_ref, a_ref, bb_ref):
        xv = x_ref[...]
        a_ref[...] = jnp.dot(xv, wa_ref[...], preferred_element_type=jnp.float32)
        bb_ref[...] = (jnp.dot(xv, wb_ref[...], preferred_element_type=jnp.float32)
                       + b1_ref[...])
    return pl.pallas_call(
        body,
        out_shape=(jax.ShapeDtypeStruct((N_NODES, D), jnp.float32),
                   jax.ShapeDtypeStruct((N_NODES, D), jnp.float32)),
    )(x, W1a, W1b, b1.reshape(1, D))


_EBLK = 8000


def _edge_matmul(ea, W1c):
    def body(ea_ref, w_ref, c_ref):
        c_ref[...] = jnp.dot(ea_ref[...], w_ref[...],
                             preferred_element_type=jnp.float32)
    return pl.pallas_call(
        body,
        grid=(N_EDGES // _EBLK,),
        in_specs=[pl.BlockSpec((_EBLK, D_EDGE), lambda i: (i, 0)),
                  pl.BlockSpec((D_EDGE, D), lambda i: (0, 0))],
        out_specs=pl.BlockSpec((_EBLK, D), lambda i: (i, 0)),
        out_shape=jax.ShapeDtypeStruct((N_EDGES, D), jnp.float32),
    )(ea, W1c)


_NBLK = 2000


def _final_matmul(S, deg, W2, b2):
    def body(s_ref, d_ref, w_ref, b2_ref, o_ref):
        o_ref[...] = (jnp.dot(s_ref[...], w_ref[...],
                              preferred_element_type=jnp.float32)
                      + d_ref[...] * b2_ref[...])
    return pl.pallas_call(
        body,
        grid=(N_NODES // _NBLK,),
        in_specs=[pl.BlockSpec((_NBLK, D), lambda i: (i, 0)),
                  pl.BlockSpec((_NBLK, 1), lambda i: (i, 0)),
                  pl.BlockSpec((D, D), lambda i: (0, 0)),
                  pl.BlockSpec((1, D), lambda i: (0, 0))],
        out_specs=pl.BlockSpec((_NBLK, D), lambda i: (i, 0)),
        out_shape=jax.ShapeDtypeStruct((N_NODES, D), jnp.float32),
    )(S, deg, W2, b2.reshape(1, D))


def _sc_gather_relu_scatter(A, Bb, C, src, dst):
    mesh = plsc.VectorSubcoreMesh(core_axis_name="c", subcore_axis_name="s")

    @functools.partial(
        pl.kernel,
        mesh=mesh,
        out_type=(jax.ShapeDtypeStruct((N_NODES, D), jnp.float32),
                  jax.ShapeDtypeStruct((NC, DEG_ROWS, D), jnp.float32)),
        scratch_types=(
            [pltpu.VMEM((CHUNK,), jnp.int32)] * 4 +
            [pltpu.VMEM((CHUNK,), jnp.int32)] * 6 +
            [pltpu.VMEM((CHUNK, D), jnp.float32)] * 8 +
            [pltpu.VMEM_SHARED((ACC_ROWS, D), jnp.float32)] +
            [pltpu.SemaphoreType.DMA] * 4
        ),
    )
    def k(a_hbm, b_hbm, c_hbm, src_hbm, dst_hbm, out_hbm, deg_hbm,
          src0, src1, dst0, dst1, sx0, sx1, ib0, ib1, po0, po1,
          a0, a1, b0, b1, c0, c1, m0, m1, s_sh, gin0, gin1, sca0, sca1):
        cid = lax.axis_index("c")
        sid = lax.axis_index("s")
        sets = ((src0, dst0, sx0, ib0, po0, a0, b0, c0, m0, gin0, sca0),
                (src1, dst1, sx1, ib1, po1, a1, b1, c1, m1, gin1, sca1))

        zero16 = jnp.zeros((NL,), jnp.float32)
        lane = lax.iota(jnp.int32, NL)
        lanes = [lane + NL * j for j in range(D // NL)]

        def zrow_body(e, carry):
            for j in range(D // NL):
                m0[e, pl.ds(j * NL, NL)] = zero16
            return carry
        lax.fori_loop(0, CHUNK, zrow_body, 0)
        for off, rows in ((0, 80), (80, 80), (160, 80), (240, 75)):
            pltpu.sync_copy(m0.at[pl.ds(0, rows)],
                            s_sh.at[pl.ds(sid * 315 + off, rows)])

        plsc.subcore_barrier()

        lo = cid * HALF
        lo2 = (1 - cid) * HALF

        def chunk_base(ch):
            return (sid + ch * NS) * CHUNK

        def fetch(ch, bufs):
            src_v, dst_v, _, _, _, a_v, b_v, c_v, _, gin, _ = bufs
            base = chunk_base(ch)
            pltpu.sync_copy(src_hbm.at[pl.ds(base, CHUNK)], src_v)
            pltpu.sync_copy(dst_hbm.at[pl.ds(base, CHUNK)], dst_v)
            pltpu.async_copy(a_hbm.at[src_v], a_v, gin)
            pltpu.async_copy(b_hbm.at[dst_v], b_v, gin)
            pltpu.async_copy(c_hbm.at[pl.ds(base, CHUNK)], c_v, gin)

        fetch(0, sets[0])

        def loop_body(kk, carry):
            for b in range(2):
                ch = 2 * kk + b
                (src_v, dst_v, sidx_v, inb_v, pos_v,
                 a_v, b_v, c_v, m_v, gin, sca) = sets[b]
                nxt = sets[1 - b]

                @pl.when(ch < CH_PER_TILE - 1)
                def _():
                    fetch(ch + 1, nxt)

                base = chunk_base(ch)
                pltpu.make_async_copy(a_hbm.at[src_v], a_v, gin).wait()
                pltpu.make_async_copy(b_hbm.at[dst_v], b_v, gin).wait()
                pltpu.make_async_copy(
                    c_hbm.at[pl.ds(base, CHUNK)], c_v, gin).wait()

                @pl.when(kk > 0)
                def _():
                    pltpu.make_async_copy(m_v, s_sh.at[sidx_v], sca).wait()

                for g in range(CHUNK // NL):
                    d16 = dst_v[pl.ds(g * NL, NL)]
                    r16 = d16 - lo
                    in_half = (r16 >= 0) & (r16 < HALF)
                    r2 = d16 - lo2
                    sidx_v[pl.ds(g * NL, NL)] = jnp.where(
                        in_half, r16, HALF + lax.shift_right_logical(r2, 7))
                    inb_v[pl.ds(g * NL, NL)] = jnp.where(in_half, 1, 0)
                    pos_v[pl.ds(g * NL, NL)] = lax.bitwise_and(r2, 127)

                def group_body(g, c2):
                    f16 = inb_v[pl.ds(g * NL, NL)]
                    p16 = pos_v[pl.ds(g * NL, NL)]
                    for l in range(NL):
                        e = g * NL + l
                        flag = f16[l] > 0
                        pos = p16[l]
                        for j in range(D // NL):
                            sl = pl.ds(j * NL, NL)
                            msg = jnp.maximum(
                                a_v[e, sl] + b_v[e, sl] + c_v[e, sl], 0.0)
                            oh = jnp.where(lanes[j] == pos, 1.0, 0.0)
                            m_v[e, sl] = jnp.where(flag, msg, oh)
                    return c2
                lax.fori_loop(0, CHUNK // NL, group_body, 0)

                pltpu.async_copy(m_v, s_sh.at[sidx_v], sca, add=True)
            return carry
        lax.fori_loop(0, CH_PER_TILE // 2, loop_body, 0)

        for b in range(2):
            (_, _, sidx_v, _, _, _, _, _, m_v, _, sca) = sets[b]
            pltpu.make_async_copy(m_v, s_sh.at[sidx_v], sca).wait()

        plsc.subcore_barrier()

        @pl.when(sid == 0)
        def _():
            pltpu.sync_copy(s_sh.at[pl.ds(0, HALF)],
                            out_hbm.at[pl.ds(cid * HALF, HALF)])
            pltpu.sync_copy(s_sh.at[pl.ds(HALF, DEG_ROWS)], deg_hbm.at[cid])

    return k(A, Bb, C, src, dst)


def kernel(x, edge_index, edge_attr, W1, b1, W2, b2):
    src = edge_index[0].astype(jnp.int32)
    dst = edge_index[1].astype(jnp.int32)
    W1a = W1[:D]
    W1b = W1[D:2 * D]
    W1c = W1[2 * D:]

    A, Bb = _node_matmul(x, W1a, W1b, b1)
    C = _edge_matmul(edge_attr, W1c)
    S, degr = _sc_gather_relu_scatter(A, Bb, C, src, dst)

    deg = jnp.concatenate([degr[1].reshape(-1)[:HALF],
                           degr[0].reshape(-1)[:HALF]])
    return _final_matmul(S, deg.reshape(N_NODES, 1), W2, b2)
